# Initial kernel scaffold; baseline (speedup 1.0000x reference)
import jax
import jax.numpy as jnp
from jax.experimental import pallas as pl


def kernel(x, edge_index, W1, b1, W2, b2):
    raise NotImplementedError("write your pallas kernel here")


# speedup vs baseline: 5.2643x; 5.2643x over previous
import functools

import jax
import jax.numpy as jnp
from jax import lax
from jax.experimental import pallas as pl
from jax.experimental.pallas import tpu as pltpu
from jax.experimental.pallas import tpu_sc as plsc

_NC = 2
_NS = 16
_C = 80
_CW = 8


@functools.lru_cache(maxsize=None)
def _make_agg(N, D, E):
    H = D // 2
    assert E % (_NS * _C) == 0
    nch = E // (_NS * _C)
    assert N % 80 == 0
    wb_rows = N // 10

    mesh = plsc.VectorSubcoreMesh(
        core_axis_name="c", subcore_axis_name="s",
        num_cores=_NC, num_subcores=_NS)

    @functools.partial(
        pl.kernel,
        out_type=(
            jax.ShapeDtypeStruct((_NC, N, H), jnp.float32),
            jax.ShapeDtypeStruct((N, _CW), jnp.float32),
        ),
        mesh=mesh,
        scratch_types=[
            pltpu.VMEM((nch, _C), jnp.int32),
            pltpu.VMEM((nch, _C), jnp.int32),
            pltpu.VMEM((_C, H), jnp.float32),
            pltpu.VMEM((_C, _CW), jnp.float32),
            pltpu.VMEM_SHARED((N, H), jnp.float32),
            pltpu.VMEM_SHARED((N, _CW), jnp.float32),
            pltpu.SemaphoreType.DMA,
        ],
        compiler_params=pltpu.CompilerParams(use_tc_tiling_on_sc=False),
    )
    def agg(x2_hbm, src_hbm, dst_hbm, zd_hbm, zc_hbm, ones_hbm,
            outs_hbm, outc_hbm,
            sidx, didx, rows, ones, acc, cacc, sem):
        c = lax.axis_index("c")
        s = lax.axis_index("s")

        @pl.when(s == 0)
        def _():
            pltpu.sync_copy(zd_hbm, acc)
            pltpu.sync_copy(zc_hbm, cacc)

        plsc.subcore_barrier()

        pltpu.sync_copy(ones_hbm, ones)

        pltpu.sync_copy(src_hbm.at[s], sidx)
        pltpu.sync_copy(dst_hbm.at[s], didx)
        off = c * N

        @pl.loop(0, nch)
        def _(j):
            for k in range(_C // 16):
                sl = pl.ds(k * 16, 16)
                sidx[j, sl] = sidx[j, sl] + off

        @pl.loop(0, nch)
        def _(j):
            pltpu.async_copy(x2_hbm.at[sidx.at[j]], rows, sem).wait()
            pltpu.sync_copy(rows, acc.at[didx.at[j]], add=True)

            @pl.when(c == 0)
            def _():
                pltpu.sync_copy(ones, cacc.at[didx.at[j]], add=True)

        plsc.subcore_barrier()

        @pl.when(s < 10)
        def _():
            r0 = s * wb_rows
            pltpu.sync_copy(acc.at[pl.ds(r0, wb_rows)],
                            outs_hbm.at[c, pl.ds(r0, wb_rows)])

            @pl.when(c == 0)
            def _():
                pltpu.sync_copy(cacc.at[pl.ds(r0, wb_rows)],
                                outc_hbm.at[pl.ds(r0, wb_rows)])

    return agg


@functools.lru_cache(maxsize=None)
def _make_dense(N, D, split_out, R=1000):
    assert N % R == 0
    H = D // 2

    def body(xa_ref, xb_ref, s_ref, cnt_ref, w_ref, b_ref, *o_refs):
        cnt = cnt_ref[:, 0:1]
        inv = jnp.where(cnt > 0.0, 1.0 / jnp.maximum(cnt, 1.0), 0.0)
        ha = xa_ref[...] + s_ref[0] * inv
        hb = xb_ref[...] + s_ref[1] * inv
        h = jnp.concatenate([ha, hb], axis=1)
        y = lax.dot_general(h, w_ref[...], (((1,), (1,)), ((), ())),
                            preferred_element_type=jnp.float32)
        y = jnp.maximum(y + b_ref[...], 0.0)
        if split_out:
            o_refs[0][...] = y[:, :H]
            o_refs[1][...] = y[:, H:]
        else:
            o_refs[0][...] = y

    if split_out:
        out_specs = [pl.BlockSpec((R, H), lambda i: (i, 0)),
                     pl.BlockSpec((R, H), lambda i: (i, 0))]
        out_shape = [jax.ShapeDtypeStruct((N, H), jnp.float32)] * 2
    else:
        out_specs = [pl.BlockSpec((R, D), lambda i: (i, 0))]
        out_shape = [jax.ShapeDtypeStruct((N, D), jnp.float32)]

    call = pl.pallas_call(
        body,
        grid=(N // R,),
        in_specs=[
            pl.BlockSpec((R, H), lambda i: (i, 0)),
            pl.BlockSpec((R, H), lambda i: (i, 0)),
            pl.BlockSpec((2, R, H), lambda i: (0, i, 0)),
            pl.BlockSpec((R, _CW), lambda i: (i, 0)),
            pl.BlockSpec((D, D), lambda i: (0, 0)),
            pl.BlockSpec((1, D), lambda i: (0, 0)),
        ],
        out_specs=out_specs,
        out_shape=out_shape,
    )
    return call


def kernel(x, edge_index, W1, b1, W2, b2):
    N, D = x.shape
    E = edge_index.shape[1]
    H = D // 2
    nch = E // (_NS * _C)
    src = edge_index[0].astype(jnp.int32).reshape(_NS, nch, _C)
    dst = edge_index[1].astype(jnp.int32).reshape(_NS, nch, _C)
    zd = jnp.zeros((N, H), jnp.float32)
    zc = jnp.zeros((N, _CW), jnp.float32)
    ones = jnp.ones((_C, _CW), jnp.float32)

    agg = _make_agg(N, D, E)
    dense_split = _make_dense(N, D, True)
    dense_full = _make_dense(N, D, False)
    b1r = b1.reshape(1, D)
    b2r = b2.reshape(1, D)

    xa, xb = x[:, :H], x[:, H:]
    x2 = jnp.concatenate([xa, xb], axis=0)
    s1, c1 = agg(x2, src, dst, zd, zc, ones)
    ha, hb = dense_split(xa, xb, s1, c1, W1, b1r)
    h2 = jnp.concatenate([ha, hb], axis=0)
    s2, _ = agg(h2, src, dst, zd, zc, ones)
    (z,) = dense_full(ha, hb, s2, c1, W2, b2r)
    return z


# speedup vs baseline: 10.3548x; 1.9670x over previous
import functools

import jax
import jax.numpy as jnp
from jax import lax
from jax.experimental import pallas as pl
from jax.experimental.pallas import tpu as pltpu
from jax.experimental.pallas import tpu_sc as plsc

_NC = 2
_NS = 16
_C = 125
_CW = 8


@functools.lru_cache(maxsize=None)
def _make_agg(N, D, E, with_counts):
    H = D // 2
    assert E % (_NS * _C) == 0
    nch = E // (_NS * _C)
    assert nch % 2 == 0
    assert N % 80 == 0
    wb_rows = N // 10

    mesh = plsc.VectorSubcoreMesh(
        core_axis_name="c", subcore_axis_name="s",
        num_cores=_NC, num_subcores=_NS)

    out_type = [jax.ShapeDtypeStruct((_NC, N, H), jnp.float32)]
    scratch = [
        pltpu.VMEM((nch, _C), jnp.int32),
        pltpu.VMEM((nch, _C), jnp.int32),
        pltpu.VMEM((_C, H), jnp.float32),
        pltpu.VMEM((_C, H), jnp.float32),
        pltpu.VMEM_SHARED((N, H), jnp.float32),
        pltpu.SemaphoreType.DMA,
        pltpu.SemaphoreType.DMA,
        pltpu.SemaphoreType.DMA,
    ]
    if with_counts:
        out_type.append(jax.ShapeDtypeStruct((N, _CW), jnp.float32))
        scratch += [
            pltpu.VMEM((_C, _CW), jnp.float32),
            pltpu.VMEM_SHARED((N, _CW), jnp.float32),
            pltpu.SemaphoreType.DMA,
        ]

    @functools.partial(
        pl.kernel,
        out_type=tuple(out_type),
        mesh=mesh,
        scratch_types=scratch,
        compiler_params=pltpu.CompilerParams(use_tc_tiling_on_sc=False),
    )
    def agg(x2_hbm, src_hbm, dst_hbm, zd_hbm, *rest):
        if with_counts:
            (zc_hbm, ones_hbm, outs_hbm, outc_hbm,
             sidx, didx, rows0, rows1, acc, gsem0, gsem1, ssem,
             ones, cacc, csem) = rest
        else:
            (outs_hbm, sidx, didx, rows0, rows1, acc,
             gsem0, gsem1, ssem) = rest
        c = lax.axis_index("c")
        s = lax.axis_index("s")
        rows = (rows0, rows1)
        gsem = (gsem0, gsem1)

        @pl.when(s == 0)
        def _():
            pltpu.sync_copy(zd_hbm, acc)
            if with_counts:
                pltpu.sync_copy(zc_hbm, cacc)

        plsc.subcore_barrier()

        if with_counts:
            pltpu.sync_copy(ones_hbm, ones)

        pltpu.sync_copy(src_hbm.at[c, s], sidx)
        pltpu.sync_copy(dst_hbm.at[s], didx)

        g0 = pltpu.async_copy(x2_hbm.at[sidx.at[0]], rows0, gsem0)
        g1 = pltpu.async_copy(x2_hbm.at[sidx.at[1]], rows1, gsem1)
        del g0, g1

        @pl.loop(0, nch, step=2)
        def _(j):
            for b in range(2):
                jj = j + b
                rb, gb = rows[b], gsem[b]
                pltpu.make_async_copy(x2_hbm.at[sidx.at[jj]], rb, gb).wait()
                sc = pltpu.async_copy(rb, acc.at[didx.at[jj]], ssem,
                                      add=True)
                if with_counts:
                    @pl.when(c == 0)
                    def _():
                        pltpu.async_copy(ones, cacc.at[didx.at[jj]], csem,
                                         add=True).wait()
                sc.wait()
                @pl.when(jj + 2 < nch)
                def _():
                    pltpu.async_copy(x2_hbm.at[sidx.at[jj + 2]], rb, gb)

        plsc.subcore_barrier()

        @pl.when(s < 10)
        def _():
            r0 = s * wb_rows
            pltpu.sync_copy(acc.at[pl.ds(r0, wb_rows)],
                            outs_hbm.at[c, pl.ds(r0, wb_rows)])
            if with_counts:
                @pl.when(c == 0)
                def _():
                    pltpu.sync_copy(cacc.at[pl.ds(r0, wb_rows)],
                                    outc_hbm.at[pl.ds(r0, wb_rows)])

    return agg


@functools.lru_cache(maxsize=None)
def _make_dense(N, D, split_out, R=1000):
    assert N % R == 0
    H = D // 2

    def body(xa_ref, xb_ref, s_ref, cnt_ref, w_ref, b_ref, o_ref):
        cnt = cnt_ref[:, 0:1]
        inv = jnp.where(cnt > 0.0, 1.0 / jnp.maximum(cnt, 1.0), 0.0)
        ha = xa_ref[...] + s_ref[0] * inv
        hb = xb_ref[...] + s_ref[1] * inv
        h = jnp.concatenate([ha, hb], axis=1)
        y = lax.dot_general(h, w_ref[...], (((1,), (1,)), ((), ())),
                            preferred_element_type=jnp.float32)
        y = jnp.maximum(y + b_ref[...], 0.0)
        if split_out:
            o_ref[0] = y[:, :H]
            o_ref[1] = y[:, H:]
        else:
            o_ref[...] = y

    if split_out:
        out_specs = pl.BlockSpec((2, R, H), lambda i: (0, i, 0))
        out_shape = jax.ShapeDtypeStruct((2, N, H), jnp.float32)
    else:
        out_specs = pl.BlockSpec((R, D), lambda i: (i, 0))
        out_shape = jax.ShapeDtypeStruct((N, D), jnp.float32)

    return pl.pallas_call(
        body,
        grid=(N // R,),
        in_specs=[
            pl.BlockSpec((R, H), lambda i: (i, 0)),
            pl.BlockSpec((R, H), lambda i: (i, 0)),
            pl.BlockSpec((2, R, H), lambda i: (0, i, 0)),
            pl.BlockSpec((R, _CW), lambda i: (i, 0)),
            pl.BlockSpec((D, D), lambda i: (0, 0)),
            pl.BlockSpec((1, D), lambda i: (0, 0)),
        ],
        out_specs=out_specs,
        out_shape=out_shape,
    )


def kernel(x, edge_index, W1, b1, W2, b2):
    N, D = x.shape
    E = edge_index.shape[1]
    H = D // 2
    nch = E // (_NS * _C)
    src = edge_index[0].astype(jnp.int32).reshape(_NS, nch, _C)
    srcs = jnp.stack([src, src + N])
    dst = edge_index[1].astype(jnp.int32).reshape(_NS, nch, _C)
    zd = jnp.zeros((N, H), jnp.float32)
    zc = jnp.zeros((N, _CW), jnp.float32)
    ones = jnp.ones((_C, _CW), jnp.float32)

    agg_c = _make_agg(N, D, E, True)
    agg = _make_agg(N, D, E, False)
    dense_split = _make_dense(N, D, True)
    dense_full = _make_dense(N, D, False)
    b1r = b1.reshape(1, D)
    b2r = b2.reshape(1, D)

    xa, xb = x[:, :H], x[:, H:]
    x2 = jnp.concatenate([xa, xb], axis=0)
    s1, c1 = agg_c(x2, srcs, dst, zd, zc, ones)
    h2 = dense_split(xa, xb, s1, c1, W1, b1r)
    (s2,) = agg(h2.reshape(2 * N, H), srcs, dst, zd)
    z = dense_full(h2[0], h2[1], s2, c1, W2, b2r)
    return z


# speedup vs baseline: 12.0542x; 1.1641x over previous
import functools

import jax
import jax.numpy as jnp
from jax import lax
from jax.experimental import pallas as pl
from jax.experimental.pallas import tpu as pltpu
from jax.experimental.pallas import tpu_sc as plsc

_NC = 2
_NS = 16
_C = 125
_CW = 8


@functools.lru_cache(maxsize=None)
def _make_agg(N, D, E, with_counts):
    H = D // 2
    assert E % (_NS * _C) == 0
    nch = E // (_NS * _C)
    assert nch % 2 == 0
    assert N % 80 == 0
    wb_rows = N // 10

    mesh = plsc.VectorSubcoreMesh(
        core_axis_name="c", subcore_axis_name="s",
        num_cores=_NC, num_subcores=_NS)

    out_type = [jax.ShapeDtypeStruct((N, D), jnp.float32)]
    scratch = [
        pltpu.VMEM((nch, _C), jnp.int32),
        pltpu.VMEM((nch, _C), jnp.int32),
        pltpu.VMEM((_C, H), jnp.float32),
        pltpu.VMEM((_C, H), jnp.float32),
        pltpu.VMEM_SHARED((N, H), jnp.float32),
        pltpu.SemaphoreType.DMA,
        pltpu.SemaphoreType.DMA,
        pltpu.SemaphoreType.DMA,
    ]
    if with_counts:
        out_type.append(jax.ShapeDtypeStruct((N, _CW), jnp.float32))
        scratch += [
            pltpu.VMEM((_C, _CW), jnp.float32),
            pltpu.VMEM_SHARED((N, _CW), jnp.float32),
            pltpu.SemaphoreType.DMA,
        ]

    @functools.partial(
        pl.kernel,
        out_type=tuple(out_type),
        mesh=mesh,
        scratch_types=scratch,
        compiler_params=pltpu.CompilerParams(use_tc_tiling_on_sc=False),
    )
    def agg(x2_hbm, src_hbm, dst_hbm, zd_hbm, *rest):
        if with_counts:
            (zc_hbm, ones_hbm, outs_hbm, outc_hbm,
             sidx, didx, rows0, rows1, acc, gsem0, gsem1, ssem,
             ones, cacc, csem) = rest
        else:
            (outs_hbm, sidx, didx, rows0, rows1, acc,
             gsem0, gsem1, ssem) = rest
        c = lax.axis_index("c")
        s = lax.axis_index("s")
        rows = (rows0, rows1)
        gsem = (gsem0, gsem1)

        @pl.when(s == 0)
        def _():
            pltpu.sync_copy(zd_hbm, acc)
            if with_counts:
                @pl.when(c == 0)
                def _():
                    pltpu.sync_copy(zc_hbm, cacc)

        plsc.subcore_barrier()

        if with_counts:
            pltpu.sync_copy(ones_hbm, ones)

        pltpu.sync_copy(src_hbm.at[c, s], sidx)
        pltpu.sync_copy(dst_hbm.at[s], didx)

        pltpu.async_copy(x2_hbm.at[sidx.at[0]], rows0, gsem0)
        pltpu.async_copy(x2_hbm.at[sidx.at[1]], rows1, gsem1)

        @pl.loop(0, nch, step=2)
        def _(j):
            for b in range(2):
                jj = j + b
                rb, gb = rows[b], gsem[b]
                pltpu.make_async_copy(x2_hbm.at[sidx.at[jj]], rb, gb).wait()
                sc = pltpu.async_copy(rb, acc.at[didx.at[jj]], ssem,
                                      add=True)
                if with_counts:
                    @pl.when(c == 0)
                    def _():
                        pltpu.async_copy(ones, cacc.at[didx.at[jj]], csem,
                                         add=True).wait()
                sc.wait()
                @pl.when(jj + 2 < nch)
                def _():
                    pltpu.async_copy(x2_hbm.at[sidx.at[jj + 2]], rb, gb)

        plsc.subcore_barrier()

        @pl.when(s < 10)
        def _():
            r0 = s * wb_rows
            pltpu.sync_copy(acc.at[pl.ds(r0, wb_rows)],
                            outs_hbm.at[pl.ds(r0, wb_rows),
                                        pl.ds(c * H, H)])
            if with_counts:
                @pl.when(c == 0)
                def _():
                    pltpu.sync_copy(cacc.at[pl.ds(r0, wb_rows)],
                                    outc_hbm.at[pl.ds(r0, wb_rows)])

    return agg


@functools.lru_cache(maxsize=None)
def _make_dense(N, D, R=1000):
    assert N % R == 0

    def body(x_ref, s_ref, c_ref, w_ref, b_ref, o_ref):
        cnt = c_ref[:, 0:1]
        inv = jnp.where(cnt > 0.0, 1.0 / jnp.maximum(cnt, 1.0), 0.0)
        h = x_ref[...] + s_ref[...] * inv
        y = lax.dot_general(h, w_ref[...], (((1,), (1,)), ((), ())),
                            preferred_element_type=jnp.float32)
        o_ref[...] = jnp.maximum(y + b_ref[...], 0.0)

    return pl.pallas_call(
        body,
        grid=(N // R,),
        in_specs=[
            pl.BlockSpec((R, D), lambda i: (i, 0)),
            pl.BlockSpec((R, D), lambda i: (i, 0)),
            pl.BlockSpec((R, _CW), lambda i: (i, 0)),
            pl.BlockSpec((D, D), lambda i: (0, 0)),
            pl.BlockSpec((1, D), lambda i: (0, 0)),
        ],
        out_specs=pl.BlockSpec((R, D), lambda i: (i, 0)),
        out_shape=jax.ShapeDtypeStruct((N, D), jnp.float32),
    )


def kernel(x, edge_index, W1, b1, W2, b2):
    N, D = x.shape
    E = edge_index.shape[1]
    nch = E // (_NS * _C)
    src = edge_index[0].astype(jnp.int32).reshape(_NS, nch, _C)
    srcs = jnp.stack([2 * src, 2 * src + 1])
    dst = edge_index[1].astype(jnp.int32).reshape(_NS, nch, _C)
    zd = jnp.zeros((N, D // 2), jnp.float32)
    zc = jnp.zeros((N, _CW), jnp.float32)
    ones = jnp.ones((_C, _CW), jnp.float32)

    agg_c = _make_agg(N, D, E, True)
    agg = _make_agg(N, D, E, False)
    dense = _make_dense(N, D)
    b1r = b1.reshape(1, D)
    b2r = b2.reshape(1, D)

    s1, c1 = agg_c(x.reshape(2 * N, D // 2), srcs, dst, zd, zc, ones)
    h = dense(x, s1, c1, W1, b1r)
    (s2,) = agg(h.reshape(2 * N, D // 2), srcs, dst, zd)
    z = dense(h, s2, c1, W2, b2r)
    return z
